# Initial kernel scaffold; baseline (speedup 1.0000x reference)
import jax
import jax.numpy as jnp
from jax.experimental import pallas as pl


def kernel(x, table):
    raise NotImplementedError("write your pallas kernel here")


# speedup vs baseline: 1.9200x; 1.9200x over previous
import jax
import jax.numpy as jnp
from jax.experimental import pallas as pl


BLK_L = 1024


def _copy_kernel(table_ref, out_ref):
    out_ref[0, :, :] = table_ref[:, :]


def kernel(x, table):
    N, L, D = x.shape
    grid = (L // BLK_L, N)
    return pl.pallas_call(
        _copy_kernel,
        grid=grid,
        in_specs=[
            pl.BlockSpec((BLK_L, D), lambda i, n: (i, 0)),
        ],
        out_specs=pl.BlockSpec((1, BLK_L, D), lambda i, n: (n, i, 0)),
        out_shape=jax.ShapeDtypeStruct((N, L, D), table.dtype),
    )(table)
